# Initial kernel scaffold; baseline (speedup 1.0000x reference)
import jax
import jax.numpy as jnp
from jax.experimental import pallas as pl


def kernel(x, edge_index, W1, a1_src, a1_dst, b1, g1, be1, W2, a2_src, a2_dst, b2, g2, be2, W3, a3_src, a3_dst, b3):
    raise NotImplementedError("write your pallas kernel here")


# speedup vs baseline: 1.1298x; 1.1298x over previous
import functools

import jax
import jax.numpy as jnp
from jax.experimental import pallas as pl

_N = 10000
_NEG_SLOPE = 0.2


def _matmul_body(x_ref, w_ref, o_ref):
    o_ref[...] = jnp.dot(x_ref[...], w_ref[...],
                         preferred_element_type=jnp.float32)


def _dense_matmul(x, w, block_rows=1000):
    n, d = x.shape
    _, f = w.shape
    grid = (n // block_rows,)
    return pl.pallas_call(
        _matmul_body,
        grid=grid,
        in_specs=[
            pl.BlockSpec((block_rows, d), lambda i: (i, 0)),
            pl.BlockSpec((d, f), lambda i: (0, 0)),
        ],
        out_specs=pl.BlockSpec((block_rows, f), lambda i: (i, 0)),
        out_shape=jax.ShapeDtypeStruct((n, f), jnp.float32),
    )(x, w)


def _gat_conv(x, src, dst, W, a_src, a_dst, b):
    H, C = a_src.shape
    h = _dense_matmul(x, W).reshape(_N, H, C)
    alpha_s = (h * a_src[None]).sum(-1)
    alpha_d = (h * a_dst[None]).sum(-1)
    e = jax.nn.leaky_relu(alpha_s[src] + alpha_d[dst], _NEG_SLOPE)
    ex = jnp.exp(e)
    den = jax.ops.segment_sum(ex, dst, num_segments=_N)
    num = jax.ops.segment_sum(h[src] * ex[:, :, None], dst, num_segments=_N)
    out = num / (den[:, :, None] + 1e-16)
    return out.reshape(_N, H * C) + b


def _batchnorm(x, g, b):
    m = x.mean(axis=0)
    v = x.var(axis=0)
    return (x - m) / jnp.sqrt(v + 1e-5) * g + b


def kernel(x, edge_index, W1, a1_src, a1_dst, b1, g1, be1, W2, a2_src, a2_dst, b2, g2, be2, W3, a3_src, a3_dst, b3):
    loop = jnp.arange(_N, dtype=edge_index.dtype)
    src = jnp.concatenate([edge_index[0], loop])
    dst = jnp.concatenate([edge_index[1], loop])
    h = jax.nn.relu(_gat_conv(x, src, dst, W1, a1_src, a1_dst, b1))
    h = _batchnorm(h, g1, be1)
    h = jax.nn.relu(_gat_conv(h, src, dst, W2, a2_src, a2_dst, b2))
    h = _batchnorm(h, g2, be2)
    h = _gat_conv(h, src, dst, W3, a3_src, a3_dst, b3)
    return jax.nn.log_softmax(h, axis=1)


# speedup vs baseline: 12.8020x; 11.3317x over previous
import functools

import jax
import jax.numpy as jnp
from jax import lax
from jax.experimental import pallas as pl
from jax.experimental.pallas import tpu as pltpu
from jax.experimental.pallas import tpu_sc as plsc

_N = 10000
_E = 320000
_NE = _E + _N
_NP = 10240
_SPAN = _NP // 16
_BLK = 128
_CHUNK = 10368
_EP = 32 * _CHUNK
_NBLK = _CHUNK // _BLK
_NEG_SLOPE = 0.2


def _matmul_body(x_ref, w_ref, o_ref):
    o_ref[...] = jnp.dot(x_ref[...], w_ref[...],
                         preferred_element_type=jnp.float32)


def _dense_matmul(x, w, block_rows=1000):
    n, d = x.shape
    _, f = w.shape
    grid = (n // block_rows,)
    return pl.pallas_call(
        _matmul_body,
        grid=grid,
        in_specs=[
            pl.BlockSpec((block_rows, d), lambda i: (i, 0)),
            pl.BlockSpec((d, f), lambda i: (0, 0)),
        ],
        out_specs=pl.BlockSpec((block_rows, f), lambda i: (i, 0)),
        out_shape=jax.ShapeDtypeStruct((n, f), jnp.float32),
    )(x, w)


@functools.lru_cache(maxsize=None)
def _make_edge_kernel(H, CP):
    mesh = plsc.VectorSubcoreMesh(core_axis_name="c", subcore_axis_name="s")

    @functools.partial(
        pl.kernel, mesh=mesh,
        compiler_params=pltpu.CompilerParams(needs_layout_passes=False,
                                             use_tc_tiling_on_sc=False),
        out_type=jax.ShapeDtypeStruct((2 * H * _NP, CP), jnp.float32),
        scratch_types=[
            pltpu.VMEM((_BLK,), jnp.int32),
            pltpu.VMEM((_BLK,), jnp.int32),
            pltpu.VMEM((_BLK,), jnp.int32),
            pltpu.VMEM((_BLK,), jnp.float32),
            pltpu.VMEM((_N,), jnp.float32),
            pltpu.VMEM((_N,), jnp.float32),
            pltpu.VMEM((_BLK, CP), jnp.float32),
            pltpu.VMEM((_SPAN, CP), jnp.float32),
            pltpu.VMEM_SHARED((_NP, CP), jnp.float32),
            pltpu.SemaphoreType.DMA,
        ],
    )
    def edge_kernel(tables, alpha_s, alpha_d, srcp, dstp, zeros_h, out,
                    src_blk, dst_blk, gidx_blk, ex_blk, as_v, ad_v,
                    rows_v, zb_v, acc, sem):
        c = lax.axis_index("c")
        s = lax.axis_index("s")
        wid = s * 2 + c
        tbase = wid * _CHUNK

        def head_pass(h, carry):
            pltpu.sync_copy(zeros_h, zb_v)
            pltpu.sync_copy(zb_v, acc.at[pl.ds(s * _SPAN, _SPAN)])
            plsc.subcore_barrier()

            pltpu.sync_copy(alpha_s.at[pl.ds(h * _N, _N)], as_v)
            pltpu.sync_copy(alpha_d.at[pl.ds(h * _N, _N)], ad_v)

            def blk_body(b, carry2):
                base = tbase + b * _BLK
                pltpu.sync_copy(srcp.at[pl.ds(base, _BLK)], src_blk)
                pltpu.sync_copy(dstp.at[pl.ds(base, _BLK)], dst_blk)
                for g in range(_BLK // 16):
                    s16 = src_blk[pl.ds(g * 16, 16)]
                    d16 = dst_blk[pl.ds(g * 16, 16)]
                    gidx_blk[pl.ds(g * 16, 16)] = s16 + h * _N
                    e = (plsc.load_gather(as_v, [s16])
                         + plsc.load_gather(ad_v, [d16]))
                    e = jnp.where(e >= 0.0, e, _NEG_SLOPE * e)
                    ex = jnp.exp(e)
                    eid = base + g * 16 + lax.iota(jnp.int32, 16)
                    ex = jnp.where(eid < _NE, ex, 0.0)
                    ex_blk[pl.ds(g * 16, 16)] = ex
                pltpu.async_copy(tables.at[gidx_blk], rows_v, sem).wait()
                for g in range(_BLK // 16):
                    exv = ex_blk[pl.ds(g * 16, 16)]
                    ridx = g * 16 + lax.iota(jnp.int32, 16)

                    def col_body(cc, carry3, exv=exv, ridx=ridx):
                        cid = jnp.full((16,), cc, dtype=jnp.int32)
                        v = plsc.load_gather(rows_v, [ridx, cid])
                        plsc.store_scatter(rows_v, [ridx, cid], v * exv)
                        return carry3

                    lax.fori_loop(0, CP, col_body, 0)
                pltpu.sync_copy(rows_v, acc.at[dst_blk], add=True)
                return carry2

            lax.fori_loop(0, _NBLK, blk_body, 0)
            plsc.subcore_barrier()

            pltpu.sync_copy(acc.at[pl.ds(s * _SPAN, _SPAN)], zb_v)
            row_off = (c * H + h) * _NP + s * _SPAN
            pltpu.sync_copy(zb_v, out.at[pl.ds(row_off, _SPAN)])
            return carry

        lax.fori_loop(0, H, head_pass, 0)

    return edge_kernel


def _gat_conv(x, srcp, dstp, W, a_src, a_dst, b):
    H, C = a_src.shape
    CP = ((C + 1 + 7) // 8) * 8
    h = _dense_matmul(x, W)
    hr = h.reshape(_N, H, C)
    alpha_s = jnp.einsum("nhc,hc->hn", hr, a_src).reshape(-1)
    alpha_d = jnp.einsum("nhc,hc->hn", hr, a_dst).reshape(-1)
    ones = jnp.ones((_N, H, 1), jnp.float32)
    padz = jnp.zeros((_N, H, CP - C - 1), jnp.float32)
    tables = jnp.concatenate([hr, ones, padz], -1)
    tables = tables.transpose(1, 0, 2).reshape(H * _N, CP)
    zeros_h = jnp.zeros((_SPAN, CP), jnp.float32)
    outp = _make_edge_kernel(H, CP)(tables, alpha_s, alpha_d,
                                    srcp, dstp, zeros_h)
    summed = outp.reshape(2, H, _NP, CP).sum(0)
    num = summed[:, :_N, :C]
    den = summed[:, :_N, C]
    res = num / (den[:, :, None] + 1e-16)
    return res.transpose(1, 0, 2).reshape(_N, H * C) + b


def _batchnorm(x, g, b):
    m = x.mean(axis=0)
    v = x.var(axis=0)
    return (x - m) / jnp.sqrt(v + 1e-5) * g + b


def kernel(x, edge_index, W1, a1_src, a1_dst, b1, g1, be1, W2, a2_src, a2_dst, b2, g2, be2, W3, a3_src, a3_dst, b3):
    loop = jnp.arange(_N, dtype=jnp.int32)
    src = jnp.concatenate([edge_index[0].astype(jnp.int32), loop])
    dst = jnp.concatenate([edge_index[1].astype(jnp.int32), loop])
    padz = jnp.zeros((_EP - _NE,), jnp.int32)
    srcp = jnp.concatenate([src, padz])
    dstp = jnp.concatenate([dst, padz])
    h = jax.nn.relu(_gat_conv(x, srcp, dstp, W1, a1_src, a1_dst, b1))
    h = _batchnorm(h, g1, be1)
    h = jax.nn.relu(_gat_conv(h, srcp, dstp, W2, a2_src, a2_dst, b2))
    h = _batchnorm(h, g2, be2)
    h = _gat_conv(h, srcp, dstp, W3, a3_src, a3_dst, b3)
    return jax.nn.log_softmax(h, axis=1)


# speedup vs baseline: 20.3419x; 1.5890x over previous
import functools

import jax
import jax.numpy as jnp
from jax import lax
from jax.experimental import pallas as pl
from jax.experimental.pallas import tpu as pltpu
from jax.experimental.pallas import tpu_sc as plsc

_N = 10000
_E = 320000
_NE = _E + _N
_NP = 10032
_SPAN = _NP // 16
_BLK = 128
_CHUNK = 10368
_EP = 32 * _CHUNK
_NBLK = _CHUNK // _BLK
_NEG_SLOPE = 0.2


def _matmul_body(x_ref, w_ref, o_ref):
    o_ref[...] = jnp.dot(x_ref[...], w_ref[...],
                         preferred_element_type=jnp.float32)


def _dense_matmul(x, w, block_rows=1000):
    n, d = x.shape
    _, f = w.shape
    grid = (n // block_rows,)
    return pl.pallas_call(
        _matmul_body,
        grid=grid,
        in_specs=[
            pl.BlockSpec((block_rows, d), lambda i: (i, 0)),
            pl.BlockSpec((d, f), lambda i: (0, 0)),
        ],
        out_specs=pl.BlockSpec((block_rows, f), lambda i: (i, 0)),
        out_shape=jax.ShapeDtypeStruct((n, f), jnp.float32),
    )(x, w)


@functools.lru_cache(maxsize=None)
def _make_edge_kernel(H, CP):
    mesh = plsc.VectorSubcoreMesh(core_axis_name="c", subcore_axis_name="s")

    @functools.partial(
        pl.kernel, mesh=mesh,
        compiler_params=pltpu.CompilerParams(needs_layout_passes=False,
                                             use_tc_tiling_on_sc=False),
        out_type=jax.ShapeDtypeStruct((2 * H * _NP, CP), jnp.float32),
        scratch_types=[
            pltpu.VMEM((_BLK,), jnp.int32),
            pltpu.VMEM((_BLK,), jnp.int32),
            pltpu.VMEM((_BLK,), jnp.int32),
            pltpu.VMEM((_N,), jnp.float32),
            pltpu.VMEM((_N,), jnp.float32),
            pltpu.VMEM((_BLK, CP), jnp.float32),
            pltpu.VMEM((_SPAN, CP), jnp.float32),
            pltpu.VMEM_SHARED((_NP, CP), jnp.float32),
            pltpu.SemaphoreType.DMA,
        ],
    )
    def edge_kernel(tables, alpha_s, alpha_d, srcp, dstp, zeros_h, out,
                    src_blk, dst_blk, gidx_blk, as_v, ad_v,
                    rows_v, zb_v, acc, sem):
        c = lax.axis_index("c")
        s = lax.axis_index("s")
        wid = s * 2 + c
        tbase = wid * _CHUNK

        def head_pass(h, carry):
            pltpu.sync_copy(zeros_h, zb_v)
            pltpu.sync_copy(zb_v, acc.at[pl.ds(s * _SPAN, _SPAN)])
            plsc.subcore_barrier()

            pltpu.sync_copy(alpha_s.at[pl.ds(h * _N, _N)], as_v)
            pltpu.sync_copy(alpha_d.at[pl.ds(h * _N, _N)], ad_v)

            def blk_body(b, carry2):
                base = tbase + b * _BLK
                pltpu.sync_copy(srcp.at[pl.ds(base, _BLK)], src_blk)
                pltpu.sync_copy(dstp.at[pl.ds(base, _BLK)], dst_blk)
                exvs = []
                for g in range(_BLK // 16):
                    s16 = src_blk[pl.ds(g * 16, 16)]
                    d16 = dst_blk[pl.ds(g * 16, 16)]
                    gidx_blk[pl.ds(g * 16, 16)] = s16 + h * _N
                    e = (plsc.load_gather(as_v, [s16])
                         + plsc.load_gather(ad_v, [d16]))
                    e = jnp.where(e >= 0.0, e, _NEG_SLOPE * e)
                    ex = jnp.exp(e)
                    eid = base + g * 16 + lax.iota(jnp.int32, 16)
                    exvs.append(jnp.where(eid < _NE, ex, 0.0))
                pltpu.async_copy(tables.at[gidx_blk], rows_v, sem).wait()
                dnums = lax.GatherDimensionNumbers(
                    offset_dims=(), collapsed_slice_dims=(0,),
                    start_index_map=(0,))
                for g in range(_BLK // 16):
                    exv = exvs[g]
                    for i in range(16):
                        spl = lax.gather(
                            exv, jnp.full((16, 1), i, jnp.int32),
                            dimension_numbers=dnums, slice_sizes=(1,),
                            mode=lax.GatherScatterMode.PROMISE_IN_BOUNDS)
                        ei = g * 16 + i
                        for k in range(CP // 16):
                            v = rows_v[ei, pl.ds(k * 16, 16)]
                            rows_v[ei, pl.ds(k * 16, 16)] = v * spl
                pltpu.sync_copy(rows_v, acc.at[dst_blk], add=True)
                return carry2

            lax.fori_loop(0, _NBLK, blk_body, 0)
            plsc.subcore_barrier()

            pltpu.sync_copy(acc.at[pl.ds(s * _SPAN, _SPAN)], zb_v)
            row_off = (c * H + h) * _NP + s * _SPAN
            pltpu.sync_copy(zb_v, out.at[pl.ds(row_off, _SPAN)])
            return carry

        lax.fori_loop(0, H, head_pass, 0)

    return edge_kernel


def _gat_conv(x, srcp, dstp, W, a_src, a_dst, b):
    H, C = a_src.shape
    CP = ((C + 1 + 15) // 16) * 16
    h = _dense_matmul(x, W)
    hr = h.reshape(_N, H, C)
    alpha_s = jnp.einsum("nhc,hc->hn", hr, a_src).reshape(-1)
    alpha_d = jnp.einsum("nhc,hc->hn", hr, a_dst).reshape(-1)
    ones = jnp.ones((_N, H, 1), jnp.float32)
    padz = jnp.zeros((_N, H, CP - C - 1), jnp.float32)
    tables = jnp.concatenate([hr, ones, padz], -1)
    tables = tables.transpose(1, 0, 2).reshape(H * _N, CP)
    zeros_h = jnp.zeros((_SPAN, CP), jnp.float32)
    outp = _make_edge_kernel(H, CP)(tables, alpha_s, alpha_d,
                                    srcp, dstp, zeros_h)
    summed = outp.reshape(2, H, _NP, CP).sum(0)
    num = summed[:, :_N, :C]
    den = summed[:, :_N, C]
    res = num / (den[:, :, None] + 1e-16)
    return res.transpose(1, 0, 2).reshape(_N, H * C) + b


def _batchnorm(x, g, b):
    m = x.mean(axis=0)
    v = x.var(axis=0)
    return (x - m) / jnp.sqrt(v + 1e-5) * g + b


def kernel(x, edge_index, W1, a1_src, a1_dst, b1, g1, be1, W2, a2_src, a2_dst, b2, g2, be2, W3, a3_src, a3_dst, b3):
    loop = jnp.arange(_N, dtype=jnp.int32)
    src = jnp.concatenate([edge_index[0].astype(jnp.int32), loop])
    dst = jnp.concatenate([edge_index[1].astype(jnp.int32), loop])
    padz = jnp.zeros((_EP - _NE,), jnp.int32)
    srcp = jnp.concatenate([src, padz])
    dstp = jnp.concatenate([dst, padz])
    h = jax.nn.relu(_gat_conv(x, srcp, dstp, W1, a1_src, a1_dst, b1))
    h = _batchnorm(h, g1, be1)
    h = jax.nn.relu(_gat_conv(h, srcp, dstp, W2, a2_src, a2_dst, b2))
    h = _batchnorm(h, g2, be2)
    h = _gat_conv(h, srcp, dstp, W3, a3_src, a3_dst, b3)
    return jax.nn.log_softmax(h, axis=1)
